# baseline (speedup 1.0000x reference)
import functools

import jax
import jax.numpy as jnp
from jax import lax
from jax.experimental import pallas as pl
from jax.experimental.pallas import tpu as pltpu
from jax.experimental.pallas import tpu_sc as plsc

VOCAB = 100000
EMB = 64
BATCH = 1024

_NC = 2
_NS = 16
_NW = _NC * _NS
_BPW = BATCH // _NW

_PROWS = 4000


def _pad_body(in_ref, out_ref):
    x = in_ref[...]
    out_ref[...] = jnp.concatenate([x, jnp.zeros_like(x)], axis=1)


def _pad_table(emb_table):
    return pl.pallas_call(
        _pad_body,
        grid=(VOCAB // _PROWS,),
        in_specs=[pl.BlockSpec((_PROWS, EMB), lambda i: (i, 0))],
        out_specs=pl.BlockSpec((_PROWS, 2 * EMB), lambda i: (i, 0)),
        out_shape=jax.ShapeDtypeStruct((VOCAB, 2 * EMB), jnp.float32),
        compiler_params=pltpu.CompilerParams(
            dimension_semantics=("arbitrary",),
        ),
    )(emb_table)


@functools.cache
def _make_sc_gather():
    mesh = plsc.VectorSubcoreMesh(core_axis_name="c", subcore_axis_name="s")

    @functools.partial(
        pl.kernel,
        mesh=mesh,
        out_type=jax.ShapeDtypeStruct((BATCH, 2 * EMB), jnp.float32),
        scratch_types=[
            pltpu.VMEM((_BPW,), jnp.int32),
            pltpu.VMEM((_BPW, 2 * EMB), jnp.float32),
            pltpu.SemaphoreType.DMA,
        ],
        compiler_params=pltpu.CompilerParams(use_tc_tiling_on_sc=True),
    )
    def _sc_gather(table_hbm, idx_hbm, out_hbm, idx_v, rows_v, sem):
        wid = lax.axis_index("s") * _NC + lax.axis_index("c")
        base = wid * _BPW
        pltpu.sync_copy(idx_hbm.at[pl.ds(base, _BPW)], idx_v)
        pltpu.async_copy(table_hbm.at[idx_v], rows_v, sem).wait()
        pltpu.sync_copy(rows_v, out_hbm.at[pl.ds(base, _BPW)])

    return _sc_gather


_VT = 2048


def _proj_body(rows_ref, w_ref, b_ref, out_ref):
    emb = rows_ref[:, :EMB]
    acc = lax.dot_general(
        emb,
        w_ref[...],
        (((1,), (1,)), ((), ())),
        preferred_element_type=jnp.float32,
    )
    out_ref[...] = acc + b_ref[...]


def _projection(rows128, fc_w, fc_b2d):
    nv = pl.cdiv(VOCAB, _VT)
    return pl.pallas_call(
        _proj_body,
        grid=(nv,),
        in_specs=[
            pl.BlockSpec((BATCH, 2 * EMB), lambda j: (0, 0)),
            pl.BlockSpec((_VT, EMB), lambda j: (j, 0)),
            pl.BlockSpec((1, _VT), lambda j: (0, j)),
        ],
        out_specs=pl.BlockSpec((BATCH, _VT), lambda j: (0, j)),
        out_shape=jax.ShapeDtypeStruct((BATCH, VOCAB), jnp.float32),
        compiler_params=pltpu.CompilerParams(
            dimension_semantics=("arbitrary",),
        ),
    )(rows128, fc_w, fc_b2d)


def kernel(words, emb_table, fc_w, fc_b):
    words = words.astype(jnp.int32)
    table128 = _pad_table(emb_table)
    rows128 = _make_sc_gather()(table128, words)
    return _projection(rows128, fc_w, fc_b.reshape(1, VOCAB))


# speedup vs baseline: 3.1108x; 3.1108x over previous
import functools

import jax
import jax.numpy as jnp
from jax import lax
from jax.experimental import pallas as pl
from jax.experimental.pallas import tpu as pltpu
from jax.experimental.pallas import tpu_sc as plsc

VOCAB = 100000
EMB = 64
BATCH = 1024

_NC = 2
_NS = 16
_NW = _NC * _NS
_BPW = BATCH // _NW

_PT = 2048


def _tpad_body(in_ref, out_ref):
    x = in_ref[...]
    eye = jnp.eye(EMB, dtype=jnp.float32)
    xt = lax.dot_general(
        x, eye, (((0,), (0,)), ((), ())),
        preferred_element_type=jnp.float32,
    )
    out_ref[...] = jnp.concatenate([xt, jnp.zeros_like(xt)], axis=1)


def _tpad_table(emb_t):
    return pl.pallas_call(
        _tpad_body,
        grid=(pl.cdiv(VOCAB, _PT),),
        in_specs=[pl.BlockSpec((EMB, _PT), lambda i: (0, i))],
        out_specs=pl.BlockSpec((_PT, 2 * EMB), lambda i: (i, 0)),
        out_shape=jax.ShapeDtypeStruct((VOCAB, 2 * EMB), jnp.float32),
        compiler_params=pltpu.CompilerParams(
            dimension_semantics=("arbitrary",),
        ),
    )(emb_t)


@functools.cache
def _make_sc_gather():
    mesh = plsc.VectorSubcoreMesh(core_axis_name="c", subcore_axis_name="s")

    @functools.partial(
        pl.kernel,
        mesh=mesh,
        out_type=jax.ShapeDtypeStruct((BATCH, 2 * EMB), jnp.float32),
        scratch_types=[
            pltpu.VMEM((_BPW,), jnp.int32),
            pltpu.VMEM((_BPW, 2 * EMB), jnp.float32),
            pltpu.SemaphoreType.DMA,
        ],
        compiler_params=pltpu.CompilerParams(use_tc_tiling_on_sc=True),
    )
    def _sc_gather(table_hbm, idx_hbm, out_hbm, idx_v, rows_v, sem):
        wid = lax.axis_index("s") * _NC + lax.axis_index("c")
        base = wid * _BPW
        pltpu.sync_copy(idx_hbm.at[pl.ds(base, _BPW)], idx_v)
        pltpu.async_copy(table_hbm.at[idx_v], rows_v, sem).wait()
        pltpu.sync_copy(rows_v, out_hbm.at[pl.ds(base, _BPW)])

    return _sc_gather


_VT = 2048


def _proj_body(rows_ref, w_ref, b_ref, out_ref):
    emb = rows_ref[:, :EMB]
    ones = jnp.ones((BATCH, 1), dtype=jnp.float32)
    emb_aug = jnp.concatenate([emb, ones], axis=1)
    w_aug = jnp.concatenate([w_ref[...], b_ref[...]], axis=0)
    out_ref[...] = lax.dot_general(
        w_aug, emb_aug, (((0,), (1,)), ((), ())),
        preferred_element_type=jnp.float32,
    )


def _projection(rows128, fc_w_t, fc_b2d):
    nv = pl.cdiv(VOCAB, _VT)
    return pl.pallas_call(
        _proj_body,
        grid=(nv,),
        in_specs=[
            pl.BlockSpec((BATCH, 2 * EMB), lambda j: (0, 0)),
            pl.BlockSpec((EMB, _VT), lambda j: (0, j)),
            pl.BlockSpec((1, _VT), lambda j: (0, j)),
        ],
        out_specs=pl.BlockSpec((_VT, BATCH), lambda j: (j, 0)),
        out_shape=jax.ShapeDtypeStruct((VOCAB, BATCH), jnp.float32),
        compiler_params=pltpu.CompilerParams(
            dimension_semantics=("arbitrary",),
        ),
    )(rows128, fc_w_t, fc_b2d)


def kernel(words, emb_table, fc_w, fc_b):
    words = words.astype(jnp.int32)
    emb_t = emb_table.T
    fc_w_t = fc_w.T
    table128 = _tpad_table(emb_t)
    rows128 = _make_sc_gather()(table128, words)
    out_t = _projection(rows128, fc_w_t, fc_b.reshape(1, VOCAB))
    return out_t.T


# speedup vs baseline: 3.2720x; 1.0518x over previous
import functools

import jax
import jax.numpy as jnp
from jax import lax
from jax.experimental import pallas as pl
from jax.experimental.pallas import tpu as pltpu
from jax.experimental.pallas import tpu_sc as plsc

VOCAB = 100000
EMB = 64
BATCH = 1024

_NC = 2
_NS = 16
_NW = _NC * _NS
_BPW = BATCH // _NW

_PT = 4096


def _tpad_body(in_ref, out_ref):
    x = in_ref[...]
    eye = jnp.eye(EMB, dtype=jnp.float32)
    xt = lax.dot_general(
        x, eye, (((0,), (0,)), ((), ())),
        preferred_element_type=jnp.float32,
    )
    out_ref[...] = jnp.concatenate([xt, jnp.zeros_like(xt)], axis=1)


def _tpad_table(emb_t):
    return pl.pallas_call(
        _tpad_body,
        grid=(pl.cdiv(VOCAB, _PT),),
        in_specs=[pl.BlockSpec((EMB, _PT), lambda i: (0, i))],
        out_specs=pl.BlockSpec((_PT, 2 * EMB), lambda i: (i, 0)),
        out_shape=jax.ShapeDtypeStruct((VOCAB, 2 * EMB), jnp.float32),
        compiler_params=pltpu.CompilerParams(
            dimension_semantics=("arbitrary",),
        ),
    )(emb_t)


@functools.cache
def _make_sc_gather():
    mesh = plsc.VectorSubcoreMesh(core_axis_name="c", subcore_axis_name="s")

    @functools.partial(
        pl.kernel,
        mesh=mesh,
        out_type=jax.ShapeDtypeStruct((BATCH, 2 * EMB), jnp.float32),
        scratch_types=[
            pltpu.VMEM((_BPW,), jnp.int32),
            pltpu.VMEM((_BPW, 2 * EMB), jnp.float32),
            pltpu.SemaphoreType.DMA,
        ],
        compiler_params=pltpu.CompilerParams(use_tc_tiling_on_sc=True),
    )
    def _sc_gather(table_hbm, idx_hbm, out_hbm, idx_v, rows_v, sem):
        wid = lax.axis_index("s") * _NC + lax.axis_index("c")
        base = wid * _BPW
        pltpu.sync_copy(idx_hbm.at[pl.ds(base, _BPW)], idx_v)
        pltpu.async_copy(table_hbm.at[idx_v], rows_v, sem).wait()
        pltpu.sync_copy(rows_v, out_hbm.at[pl.ds(base, _BPW)])

    return _sc_gather


_VT = 2048


def _proj_body(rows_ref, w_ref, b_ref, out_ref):
    emb = rows_ref[:, :EMB]
    ones = jnp.ones((BATCH, 1), dtype=jnp.float32)
    emb_aug = jnp.concatenate([emb, ones], axis=1)
    w_aug = jnp.concatenate([w_ref[...], b_ref[...]], axis=0)
    out_ref[...] = lax.dot_general(
        w_aug, emb_aug, (((0,), (1,)), ((), ())),
        preferred_element_type=jnp.float32,
    )


def _projection(rows128, fc_w_t, fc_b2d):
    nv = pl.cdiv(VOCAB, _VT)
    return pl.pallas_call(
        _proj_body,
        grid=(nv,),
        in_specs=[
            pl.BlockSpec((BATCH, 2 * EMB), lambda j: (0, 0)),
            pl.BlockSpec((EMB, _VT), lambda j: (0, j)),
            pl.BlockSpec((1, _VT), lambda j: (0, j)),
        ],
        out_specs=pl.BlockSpec((_VT, BATCH), lambda j: (j, 0)),
        out_shape=jax.ShapeDtypeStruct((VOCAB, BATCH), jnp.float32),
        compiler_params=pltpu.CompilerParams(
            dimension_semantics=("arbitrary",),
        ),
    )(rows128, fc_w_t, fc_b2d)


def kernel(words, emb_table, fc_w, fc_b):
    words = words.astype(jnp.int32)
    emb_t = emb_table.T
    fc_w_t = fc_w.T
    table128 = _tpad_table(emb_t)
    rows128 = _make_sc_gather()(table128, words)
    out_t = _projection(rows128, fc_w_t, fc_b.reshape(1, VOCAB))
    return out_t.T


# speedup vs baseline: 3.4202x; 1.0453x over previous
import functools

import jax
import jax.numpy as jnp
from jax import lax
from jax.experimental import pallas as pl
from jax.experimental.pallas import tpu as pltpu
from jax.experimental.pallas import tpu_sc as plsc

VOCAB = 100000
EMB = 64
BATCH = 1024

_NC = 2
_NS = 16
_NW = _NC * _NS
_BPW = BATCH // _NW

_PT = 8192


def _tpad_body(in_ref, out_ref):
    x = in_ref[...]
    eye = jnp.eye(EMB, dtype=jnp.float32)
    xt = lax.dot_general(
        x, eye, (((0,), (0,)), ((), ())),
        preferred_element_type=jnp.float32,
    )
    out_ref[...] = jnp.concatenate([xt, jnp.zeros_like(xt)], axis=1)


def _tpad_table(emb_t):
    return pl.pallas_call(
        _tpad_body,
        grid=(pl.cdiv(VOCAB, _PT),),
        in_specs=[pl.BlockSpec((EMB, _PT), lambda i: (0, i))],
        out_specs=pl.BlockSpec((_PT, 2 * EMB), lambda i: (i, 0)),
        out_shape=jax.ShapeDtypeStruct((VOCAB, 2 * EMB), jnp.float32),
        compiler_params=pltpu.CompilerParams(
            dimension_semantics=("arbitrary",),
        ),
    )(emb_t)


@functools.cache
def _make_sc_gather():
    mesh = plsc.VectorSubcoreMesh(core_axis_name="c", subcore_axis_name="s")

    @functools.partial(
        pl.kernel,
        mesh=mesh,
        out_type=jax.ShapeDtypeStruct((BATCH, 2 * EMB), jnp.float32),
        scratch_types=[
            pltpu.VMEM((_BPW,), jnp.int32),
            pltpu.VMEM((_BPW, 2 * EMB), jnp.float32),
            pltpu.SemaphoreType.DMA,
        ],
        compiler_params=pltpu.CompilerParams(use_tc_tiling_on_sc=True),
    )
    def _sc_gather(table_hbm, idx_hbm, out_hbm, idx_v, rows_v, sem):
        wid = lax.axis_index("s") * _NC + lax.axis_index("c")
        base = wid * _BPW
        pltpu.sync_copy(idx_hbm.at[pl.ds(base, _BPW)], idx_v)
        pltpu.async_copy(table_hbm.at[idx_v], rows_v, sem).wait()
        pltpu.sync_copy(rows_v, out_hbm.at[pl.ds(base, _BPW)])

    return _sc_gather


_VT = 2048


def _proj_body(rows_ref, w_ref, b_ref, out_ref):
    emb = rows_ref[:, :EMB]
    ones = jnp.ones((BATCH, 1), dtype=jnp.float32)
    emb_aug = jnp.concatenate([emb, ones], axis=1)
    w_aug = jnp.concatenate([w_ref[...], b_ref[...]], axis=0)
    out_ref[...] = lax.dot_general(
        w_aug, emb_aug, (((0,), (1,)), ((), ())),
        preferred_element_type=jnp.float32,
    )


def _projection(rows128, fc_w_t, fc_b2d):
    nv = pl.cdiv(VOCAB, _VT)
    return pl.pallas_call(
        _proj_body,
        grid=(nv,),
        in_specs=[
            pl.BlockSpec((BATCH, 2 * EMB), lambda j: (0, 0)),
            pl.BlockSpec((EMB, _VT), lambda j: (0, j)),
            pl.BlockSpec((1, _VT), lambda j: (0, j)),
        ],
        out_specs=pl.BlockSpec((_VT, BATCH), lambda j: (j, 0)),
        out_shape=jax.ShapeDtypeStruct((VOCAB, BATCH), jnp.float32),
        compiler_params=pltpu.CompilerParams(
            dimension_semantics=("arbitrary",),
        ),
    )(rows128, fc_w_t, fc_b2d)


def kernel(words, emb_table, fc_w, fc_b):
    words = words.astype(jnp.int32)
    emb_t = emb_table.T
    fc_w_t = fc_w.T
    table128 = _tpad_table(emb_t)
    rows128 = _make_sc_gather()(table128, words)
    out_t = _projection(rows128, fc_w_t, fc_b.reshape(1, VOCAB))
    return out_t.T


# speedup vs baseline: 3.5083x; 1.0258x over previous
import functools

import jax
import jax.numpy as jnp
from jax import lax
from jax.experimental import pallas as pl
from jax.experimental.pallas import tpu as pltpu
from jax.experimental.pallas import tpu_sc as plsc

VOCAB = 100000
EMB = 64
BATCH = 1024

_NC = 2
_NS = 16
_NW = _NC * _NS
_BPW = BATCH // _NW

_PT = 16384


def _tpad_body(in_ref, out_ref):
    x = in_ref[...]
    eye = jnp.eye(EMB, dtype=jnp.float32)
    xt = lax.dot_general(
        x, eye, (((0,), (0,)), ((), ())),
        preferred_element_type=jnp.float32,
    )
    out_ref[...] = jnp.concatenate([xt, jnp.zeros_like(xt)], axis=1)


def _tpad_table(emb_t):
    return pl.pallas_call(
        _tpad_body,
        grid=(pl.cdiv(VOCAB, _PT),),
        in_specs=[pl.BlockSpec((EMB, _PT), lambda i: (0, i))],
        out_specs=pl.BlockSpec((_PT, 2 * EMB), lambda i: (i, 0)),
        out_shape=jax.ShapeDtypeStruct((VOCAB, 2 * EMB), jnp.float32),
        compiler_params=pltpu.CompilerParams(
            dimension_semantics=("arbitrary",),
        ),
    )(emb_t)


@functools.cache
def _make_sc_gather():
    mesh = plsc.VectorSubcoreMesh(core_axis_name="c", subcore_axis_name="s")

    @functools.partial(
        pl.kernel,
        mesh=mesh,
        out_type=jax.ShapeDtypeStruct((BATCH, 2 * EMB), jnp.float32),
        scratch_types=[
            pltpu.VMEM((_BPW,), jnp.int32),
            pltpu.VMEM((_BPW, 2 * EMB), jnp.float32),
            pltpu.SemaphoreType.DMA,
        ],
        compiler_params=pltpu.CompilerParams(use_tc_tiling_on_sc=True),
    )
    def _sc_gather(table_hbm, idx_hbm, out_hbm, idx_v, rows_v, sem):
        wid = lax.axis_index("s") * _NC + lax.axis_index("c")
        base = wid * _BPW
        pltpu.sync_copy(idx_hbm.at[pl.ds(base, _BPW)], idx_v)
        pltpu.async_copy(table_hbm.at[idx_v], rows_v, sem).wait()
        pltpu.sync_copy(rows_v, out_hbm.at[pl.ds(base, _BPW)])

    return _sc_gather


_VT = 2048


def _proj_body(rows_ref, w_ref, b_ref, out_ref):
    emb = rows_ref[:, :EMB]
    ones = jnp.ones((BATCH, 1), dtype=jnp.float32)
    emb_aug = jnp.concatenate([emb, ones], axis=1)
    w_aug = jnp.concatenate([w_ref[...], b_ref[...]], axis=0)
    out_ref[...] = lax.dot_general(
        w_aug, emb_aug, (((0,), (1,)), ((), ())),
        preferred_element_type=jnp.float32,
    )


def _projection(rows128, fc_w_t, fc_b2d):
    nv = pl.cdiv(VOCAB, _VT)
    return pl.pallas_call(
        _proj_body,
        grid=(nv,),
        in_specs=[
            pl.BlockSpec((BATCH, 2 * EMB), lambda j: (0, 0)),
            pl.BlockSpec((EMB, _VT), lambda j: (0, j)),
            pl.BlockSpec((1, _VT), lambda j: (0, j)),
        ],
        out_specs=pl.BlockSpec((_VT, BATCH), lambda j: (j, 0)),
        out_shape=jax.ShapeDtypeStruct((VOCAB, BATCH), jnp.float32),
        compiler_params=pltpu.CompilerParams(
            dimension_semantics=("arbitrary",),
        ),
    )(rows128, fc_w_t, fc_b2d)


def kernel(words, emb_table, fc_w, fc_b):
    words = words.astype(jnp.int32)
    emb_t = emb_table.T
    fc_w_t = fc_w.T
    table128 = _tpad_table(emb_t)
    rows128 = _make_sc_gather()(table128, words)
    out_t = _projection(rows128, fc_w_t, fc_b.reshape(1, VOCAB))
    return out_t.T
